# baseline (speedup 1.0000x reference)
import functools

import jax
import jax.numpy as jnp
from jax import lax
from jax.experimental import pallas as pl
from jax.experimental.pallas import tpu as pltpu
from jax.experimental.pallas import tpu_sc as plsc

T = 2048
H = 1024
F = 4096
E = 8
BT = 128
NBLK = T * 2 // BT + E
PAD = NBLK * BT

_mesh = plsc.VectorSubcoreMesh(core_axis_name="c", subcore_axis_name="s")


def _router_body(flat_ref, wr_ref, i1_ref, i2_ref, wa_ref, wb_ref):
    l = lax.dot_general(flat_ref[...], wr_ref[...],
                        (((1,), (1,)), ((), ())),
                        preferred_element_type=jnp.float32)
    lane = lax.broadcasted_iota(jnp.int32, (T, E), 1)
    m1 = jnp.max(l, axis=1, keepdims=True)
    i1 = jnp.min(jnp.where(l >= m1, lane, E), axis=1, keepdims=True)
    l2 = jnp.where(lane == i1, jnp.float32(-1e30), l)
    m2 = jnp.max(l2, axis=1, keepdims=True)
    i2 = jnp.min(jnp.where(l2 >= m2, lane, E), axis=1, keepdims=True)
    wa = 1.0 / (1.0 + jnp.exp(m2 - m1))
    i1_ref[...] = i1
    i2_ref[...] = i2
    wa_ref[...] = wa
    wb_ref[...] = 1.0 - wa


def _run_router(flat, Wr):
    return pl.pallas_call(
        _router_body,
        out_shape=[
            jax.ShapeDtypeStruct((T, 1), jnp.int32),
            jax.ShapeDtypeStruct((T, 1), jnp.int32),
            jax.ShapeDtypeStruct((T, 1), jnp.float32),
            jax.ShapeDtypeStruct((T, 1), jnp.float32),
        ],
    )(flat, Wr)


def _dispatch_body(top1_hbm, top2_hbm, wa_hbm, wb_hbm,
                   stok_hbm, sw_hbm, pos0_hbm, pos1_hbm, btab_hbm,
                   id1_v, id2_v, wa_v, wb_v, stok_v, sw_v, pos0_v, pos1_v,
                   btab_v):
    c = lax.axis_index("c")
    s = lax.axis_index("s")

    @pl.when(jnp.logical_and(c == 0, s == 0))
    def _work():
        pltpu.sync_copy(top1_hbm, id1_v)
        pltpu.sync_copy(top2_hbm, id2_v)
        pltpu.sync_copy(wa_hbm, wa_v)
        pltpu.sync_copy(wb_hbm, wb_v)

        def cnt_body(j, acc):
            v1 = id1_v[pl.ds(j * 16, 16)]
            v2 = id2_v[pl.ds(j * 16, 16)]
            return tuple(acc[e]
                         + (v1 == e).astype(jnp.int32)
                         + (v2 == e).astype(jnp.int32)
                         for e in range(E))

        acc0 = tuple(jnp.zeros((16,), jnp.int32) for _ in range(E))
        accs = lax.fori_loop(0, T // 16, cnt_body, acc0)
        cnt = [jnp.sum(accs[e]) for e in range(E)]
        nb = [(cnt[e] + (BT - 1)) // BT for e in range(E)]
        cumnb = [jnp.int32(0)]
        for e in range(E):
            cumnb.append(cumnb[-1] + nb[e])
        off = [cumnb[e] * BT for e in range(E)]
        total_blocks = cumnb[E]

        def z_body(j, carry):
            stok_v[pl.ds(j * 16, 16)] = jnp.zeros((16,), jnp.int32)
            sw_v[pl.ds(j * 16, 16)] = jnp.zeros((16,), jnp.float32)
            return carry

        lax.fori_loop(0, PAD // 16, z_body, 0)

        for e in range(E):
            cursor = off[e]
            for ids_v, w_v, pos_v in ((id1_v, wa_v, pos0_v),
                                      (id2_v, wb_v, pos1_v)):
                def p_body(j, cur, ids_v=ids_v, w_v=w_v, pos_v=pos_v, e=e):
                    v = ids_v[pl.ds(j * 16, 16)]
                    w = w_v[pl.ds(j * 16, 16)]
                    m = v == e
                    mi = m.astype(jnp.int32)
                    rank = plsc.cumsum(mi)
                    pos = cur + rank - 1
                    tok = j * 16 + lax.iota(jnp.int32, 16)
                    plsc.store_scatter(stok_v, [pos], tok, mask=m)
                    plsc.store_scatter(sw_v, [pos], w, mask=m)
                    plsc.store_scatter(pos_v, [tok], pos, mask=m)
                    return cur + jnp.sum(mi)

                cursor = lax.fori_loop(0, T // 16, p_body, cursor)

        for j in range(48 // 16):
            gvec = j * 16 + lax.iota(jnp.int32, 16)
            ev = jnp.zeros((16,), jnp.int32)
            for e in range(1, E):
                ev = ev + (gvec >= cumnb[e]).astype(jnp.int32)
            btab_v[0, pl.ds(j * 16, 16)] = ev
            btab_v[1, pl.ds(j * 16, 16)] = jnp.minimum(gvec, total_blocks - 1)
            btab_v[2, pl.ds(j * 16, 16)] = (gvec < total_blocks).astype(jnp.int32)

        pltpu.sync_copy(stok_v, stok_hbm)
        pltpu.sync_copy(sw_v, sw_hbm)
        pltpu.sync_copy(pos0_v, pos0_hbm)
        pltpu.sync_copy(pos1_v, pos1_hbm)
        pltpu.sync_copy(btab_v, btab_hbm)


_dispatch = pl.kernel(
    _dispatch_body, mesh=_mesh,
    compiler_params=pltpu.CompilerParams(needs_layout_passes=False),
    out_type=[
        jax.ShapeDtypeStruct((PAD,), jnp.int32),
        jax.ShapeDtypeStruct((PAD,), jnp.float32),
        jax.ShapeDtypeStruct((T,), jnp.int32),
        jax.ShapeDtypeStruct((T,), jnp.int32),
        jax.ShapeDtypeStruct((3, 48), jnp.int32),
    ],
    scratch_types=[
        pltpu.VMEM((T,), jnp.int32),
        pltpu.VMEM((T,), jnp.int32),
        pltpu.VMEM((T,), jnp.float32),
        pltpu.VMEM((T,), jnp.float32),
        pltpu.VMEM((PAD,), jnp.int32),
        pltpu.VMEM((PAD,), jnp.float32),
        pltpu.VMEM((T,), jnp.int32),
        pltpu.VMEM((T,), jnp.int32),
        pltpu.VMEM((3, 48), jnp.int32),
    ],
)


_GROWS = PAD // 32
_GCH = 32


def _gather_body(stok_hbm, flat_hbm, xs_hbm, idx_v, rows_v, sem):
    c = lax.axis_index("c")
    s = lax.axis_index("s")
    wid = s * 2 + c
    base = wid * _GROWS
    for k in range(_GROWS // _GCH):
        pltpu.sync_copy(stok_hbm.at[pl.ds(base + k * _GCH, _GCH)], idx_v.at[k])
    for k in range(_GROWS // _GCH):
        pltpu.async_copy(flat_hbm.at[idx_v.at[k]], rows_v, sem).wait()
        pltpu.sync_copy(rows_v, xs_hbm.at[pl.ds(base + k * _GCH, _GCH)])


_gather = pl.kernel(
    _gather_body, mesh=_mesh,
    compiler_params=pltpu.CompilerParams(needs_layout_passes=False),
    out_type=[jax.ShapeDtypeStruct((PAD, H), jnp.float32)],
    scratch_types=[
        pltpu.VMEM((_GROWS // _GCH, _GCH), jnp.int32),
        pltpu.VMEM((_GCH, H), jnp.float32),
        pltpu.SemaphoreType.DMA,
    ],
)


def _ffn1_body(tab_ref, xs_ref, w1_ref, hmid_ref):
    g = pl.program_id(0)

    @pl.when(tab_ref[2, g] == 1)
    def _():
        h = lax.dot_general(xs_ref[...], w1_ref[0], (((1,), (1,)), ((), ())),
                            preferred_element_type=jnp.float32)
        hmid_ref[...] = h * jax.nn.sigmoid(h)


def _ffn2_body(tab_ref, hmid_ref, w2_ref, sw_ref, ys_ref):
    g = pl.program_id(0)

    @pl.when(tab_ref[2, g] == 1)
    def _():
        y = lax.dot_general(hmid_ref[...], w2_ref[0], (((1,), (1,)), ((), ())),
                            preferred_element_type=jnp.float32)
        ys_ref[...] = y * sw_ref[0, 0][:, None]


def _run_ffn(btab, xs, W1, W2, sw3):
    gs1 = pltpu.PrefetchScalarGridSpec(
        num_scalar_prefetch=1,
        grid=(NBLK,),
        in_specs=[
            pl.BlockSpec((BT, H), lambda g, tab: (tab[1, g], 0)),
            pl.BlockSpec((1, F, H), lambda g, tab: (tab[0, g], 0, 0)),
        ],
        out_specs=pl.BlockSpec((BT, F), lambda g, tab: (tab[1, g], 0)),
    )
    hmid = pl.pallas_call(
        _ffn1_body,
        grid_spec=gs1,
        out_shape=jax.ShapeDtypeStruct((PAD, F), jnp.float32),
        compiler_params=pltpu.CompilerParams(
            dimension_semantics=("arbitrary",),
            vmem_limit_bytes=100 * 1024 * 1024),
    )(btab, xs, W1)
    gs2 = pltpu.PrefetchScalarGridSpec(
        num_scalar_prefetch=1,
        grid=(NBLK,),
        in_specs=[
            pl.BlockSpec((BT, F), lambda g, tab: (tab[1, g], 0)),
            pl.BlockSpec((1, H, F), lambda g, tab: (tab[0, g], 0, 0)),
            pl.BlockSpec((1, 1, BT), lambda g, tab: (tab[1, g], 0, 0)),
        ],
        out_specs=pl.BlockSpec((BT, H), lambda g, tab: (tab[1, g], 0)),
    )
    return pl.pallas_call(
        _ffn2_body,
        grid_spec=gs2,
        out_shape=jax.ShapeDtypeStruct((PAD, H), jnp.float32),
        compiler_params=pltpu.CompilerParams(
            dimension_semantics=("arbitrary",),
            vmem_limit_bytes=100 * 1024 * 1024),
    )(btab, hmid, W2, sw3)


_CTOK = T // 32
_CCH = 16


def _combine_body(ys_hbm, pos0_hbm, pos1_hbm, out_hbm, i0_v, i1_v, a_v, b_v,
                  sem):
    c = lax.axis_index("c")
    s = lax.axis_index("s")
    wid = s * 2 + c
    base = wid * _CTOK
    nch = _CTOK // _CCH
    for k in range(nch):
        pltpu.sync_copy(pos0_hbm.at[pl.ds(base + k * _CCH, _CCH)], i0_v.at[k])
        pltpu.sync_copy(pos1_hbm.at[pl.ds(base + k * _CCH, _CCH)], i1_v.at[k])
    for k in range(nch):
        pltpu.async_copy(ys_hbm.at[i0_v.at[k]], a_v, sem).wait()
        pltpu.async_copy(ys_hbm.at[i1_v.at[k]], b_v, sem).wait()
        for r in range(_CCH):
            def add_body(j, carry, r=r):
                sl = pl.ds(j * 16, 16)
                a_v[r, sl] = a_v[r, sl] + b_v[r, sl]
                return carry
            lax.fori_loop(0, H // 16, add_body, 0)
        pltpu.sync_copy(a_v, out_hbm.at[pl.ds(base + k * _CCH, _CCH)])


_combine = pl.kernel(
    _combine_body, mesh=_mesh,
    compiler_params=pltpu.CompilerParams(needs_layout_passes=False),
    out_type=[jax.ShapeDtypeStruct((T, H), jnp.float32)],
    scratch_types=[
        pltpu.VMEM((T // 32 // _CCH, _CCH), jnp.int32),
        pltpu.VMEM((T // 32 // _CCH, _CCH), jnp.int32),
        pltpu.VMEM((_CCH, H), jnp.float32),
        pltpu.VMEM((_CCH, H), jnp.float32),
        pltpu.SemaphoreType.DMA,
    ],
)


def kernel(hidden_states, Wr, W1, W2):
    b, s, h = hidden_states.shape
    flat = hidden_states.reshape(-1, h)

    i1, i2, wa, wb = _run_router(flat, Wr)
    top1 = i1.reshape(-1)
    top2 = i2.reshape(-1)
    wa = wa.reshape(-1)
    wb = wb.reshape(-1)

    stok, sw, pos0, pos1, btab = _dispatch(top1, top2, wa, wb)
    (xs,) = _gather(stok, flat)
    ys = _run_ffn(btab, xs, W1, W2, sw.reshape(NBLK, 1, BT))
    (out,) = _combine(ys, pos0, pos1)
    return out.reshape(b, s, h)
